# baseline (speedup 1.0000x reference)
import functools

import jax
import jax.numpy as jnp
from jax import lax
from jax.experimental import pallas as pl
from jax.experimental.pallas import tpu as pltpu
from jax.experimental.pallas import tpu_sc as plsc

N = 10000
E = 320000
F0, F1, F2 = 128, 128, 2

NC = 2
NS = 16
NW = NC * NS
CHUNK = 128
NBUF = 2
GB = 8
NCHUNK = 80
NBATCH = NCHUNK // GB
EW = NCHUNK * CHUNK
RPAD = 10240
RPW = RPAD // NS
PAD_DST = N
DPAD = 8

_mesh = plsc.VectorSubcoreMesh(core_axis_name="c", subcore_axis_name="s")


def _make_deg_kernel():
    @functools.partial(
        pl.kernel,
        mesh=_mesh,
        out_type=jax.ShapeDtypeStruct((NC, RPAD, DPAD), jnp.float32),
        scratch_types=[
            pltpu.VMEM((NCHUNK, CHUNK), jnp.int32),
            pltpu.VMEM((CHUNK, DPAD), jnp.float32),
            pltpu.VMEM_SHARED((RPAD, DPAD), jnp.float32),
        ],
        compiler_params=pltpu.CompilerParams(use_tc_tiling_on_sc=False),
    )
    def deg_kernel(dsts_hbm, zeros_hbm, ones_hbm, out_hbm, idx_d, rows, acc):
        c = lax.axis_index("c")
        s = lax.axis_index("s")
        w = s * NC + c
        pltpu.sync_copy(dsts_hbm.at[w], idx_d)
        pltpu.sync_copy(zeros_hbm, rows)
        for k in range(RPW // CHUNK):
            pltpu.sync_copy(rows, acc.at[pl.ds(s * RPW + k * CHUNK, CHUNK)])
        plsc.subcore_barrier()
        pltpu.sync_copy(ones_hbm, rows)

        def body(j, carry):
            pltpu.sync_copy(rows, acc.at[idx_d.at[j]], add=True)
            return carry

        lax.fori_loop(0, NCHUNK, body, 0)
        plsc.subcore_barrier()
        pltpu.sync_copy(acc.at[pl.ds(s * RPW, RPW)],
                        out_hbm.at[c, pl.ds(s * RPW, RPW)])

    return deg_kernel


def _make_agg_kernel(D):
    @functools.partial(
        pl.kernel,
        mesh=_mesh,
        out_type=jax.ShapeDtypeStruct((NC, RPAD, D), jnp.float32),
        scratch_types=[
            pltpu.VMEM((NCHUNK, CHUNK), jnp.int32),
            pltpu.VMEM((NCHUNK, CHUNK), jnp.int32),
            pltpu.VMEM((CHUNK, D), jnp.float32),
            pltpu.VMEM_SHARED((RPAD, D), jnp.float32),
            pltpu.SemaphoreType.DMA,
        ],
        compiler_params=(None if D % 128 == 0 else
                         pltpu.CompilerParams(use_tc_tiling_on_sc=False)),
    )
    def agg_kernel(table_hbm, srcs_hbm, dsts_hbm, zeros_hbm, out_hbm,
                   idx_s, idx_d, rows, acc, sem_g):
        c = lax.axis_index("c")
        s = lax.axis_index("s")
        w = s * NC + c
        pltpu.sync_copy(srcs_hbm.at[w], idx_s)
        pltpu.sync_copy(dsts_hbm.at[w], idx_d)
        pltpu.sync_copy(zeros_hbm, rows)
        for k in range(RPW // CHUNK):
            pltpu.sync_copy(rows, acc.at[pl.ds(s * RPW + k * CHUNK, CHUNK)])
        plsc.subcore_barrier()

        def body(j, carry):
            pltpu.async_copy(table_hbm.at[idx_s.at[j]], rows, sem_g).wait()
            pltpu.sync_copy(rows, acc.at[idx_d.at[j]], add=True)
            return carry

        lax.fori_loop(0, NCHUNK, body, 0)
        plsc.subcore_barrier()
        pltpu.sync_copy(acc.at[pl.ds(s * RPW, RPW)],
                        out_hbm.at[c, pl.ds(s * RPW, RPW)])

    return agg_kernel


_deg_kernel = _make_deg_kernel()
_agg128 = _make_agg_kernel(F1)
_agg8 = _make_agg_kernel(DPAD)

BR = 400
GRID = N // BR


def _tc1_body(deg_ref, x_ref, w_ref, h_ref, n_ref):
    deg = deg_ref[0, :, 0:1] + deg_ref[1, :, 0:1]
    norm = lax.rsqrt(jnp.maximum(deg, 1.0))
    h = jnp.dot(x_ref[...], w_ref[...], preferred_element_type=jnp.float32)
    h_ref[...] = h * norm
    n_ref[...] = norm


def _tc2_body(p_ref, n_ref, b_ref, w_ref, z_ref):
    norm = n_ref[...]
    h1 = jnp.maximum((p_ref[0] + p_ref[1]) * norm + b_ref[...], 0.0)
    z = jnp.dot(h1, w_ref[...], preferred_element_type=jnp.float32)
    z_ref[...] = z * norm


def _tc3_body(q_ref, n_ref, b_ref, o_ref):
    logits = (q_ref[0, :, 0:F2] + q_ref[1, :, 0:F2]) * n_ref[...] + b_ref[...]
    m = jnp.max(logits, axis=1, keepdims=True)
    e = jnp.exp(logits - m)
    o_ref[...] = e / jnp.sum(e, axis=1, keepdims=True)


def kernel(x, edge_index, W0, b0, W1, b1):
    ei = edge_index.astype(jnp.int32)
    npad = NW * EW - E
    src = jnp.concatenate([ei[0], jnp.zeros((npad,), jnp.int32)])
    dst = jnp.concatenate([ei[1], jnp.full((npad,), PAD_DST, jnp.int32)])
    srcs = src.reshape(NW, NCHUNK, CHUNK)
    dsts = dst.reshape(NW, NCHUNK, CHUNK)

    zeros8 = jnp.zeros((CHUNK, DPAD), jnp.float32)
    ones8 = jnp.ones((CHUNK, DPAD), jnp.float32)
    zeros128 = jnp.zeros((CHUNK, F1), jnp.float32)
    W1p = jnp.pad(W1, ((0, 0), (0, DPAD - F2)))

    deg_p = _deg_kernel(dsts, zeros8, ones8)

    h_scaled, norm = pl.pallas_call(
        _tc1_body,
        grid=(GRID,),
        in_specs=[
            pl.BlockSpec((NC, BR, DPAD), lambda i: (0, i, 0)),
            pl.BlockSpec((BR, F0), lambda i: (i, 0)),
            pl.BlockSpec((F0, F1), lambda i: (0, 0)),
        ],
        out_specs=[
            pl.BlockSpec((BR, F1), lambda i: (i, 0)),
            pl.BlockSpec((BR, 1), lambda i: (i, 0)),
        ],
        out_shape=[
            jax.ShapeDtypeStruct((N, F1), jnp.float32),
            jax.ShapeDtypeStruct((N, 1), jnp.float32),
        ],
    )(deg_p, x, W0)

    p = _agg128(h_scaled, srcs, dsts, zeros128)

    z_scaled = pl.pallas_call(
        _tc2_body,
        grid=(GRID,),
        in_specs=[
            pl.BlockSpec((NC, BR, F1), lambda i: (0, i, 0)),
            pl.BlockSpec((BR, 1), lambda i: (i, 0)),
            pl.BlockSpec((1, F1), lambda i: (0, 0)),
            pl.BlockSpec((F0, DPAD), lambda i: (0, 0)),
        ],
        out_specs=pl.BlockSpec((BR, DPAD), lambda i: (i, 0)),
        out_shape=jax.ShapeDtypeStruct((N, DPAD), jnp.float32),
    )(p, norm, b0.reshape(1, F1), W1p)

    q = _agg8(z_scaled, srcs, dsts, zeros8)

    out = pl.pallas_call(
        _tc3_body,
        grid=(GRID,),
        in_specs=[
            pl.BlockSpec((NC, BR, DPAD), lambda i: (0, i, 0)),
            pl.BlockSpec((BR, 1), lambda i: (i, 0)),
            pl.BlockSpec((1, F2), lambda i: (0, 0)),
        ],
        out_specs=pl.BlockSpec((BR, F2), lambda i: (i, 0)),
        out_shape=jax.ShapeDtypeStruct((N, F2), jnp.float32),
    )(q, norm, b1.reshape(1, F2))

    return out


# speedup vs baseline: 1.0023x; 1.0023x over previous
import functools

import jax
import jax.numpy as jnp
from jax import lax
from jax.experimental import pallas as pl
from jax.experimental.pallas import tpu as pltpu
from jax.experimental.pallas import tpu_sc as plsc

N = 10000
E = 320000
F0, F1, F2 = 128, 128, 2

NC = 2
NS = 16
NW = NC * NS
CHUNK = 128
NBUF = 2
GB = 8
NCHUNK = 80
NBATCH = NCHUNK // GB
EW = NCHUNK * CHUNK
RPAD = 10240
RPW = RPAD // NS
PAD_DST = N
DPAD = 8

_mesh = plsc.VectorSubcoreMesh(core_axis_name="c", subcore_axis_name="s")


def _make_deg_kernel():
    @functools.partial(
        pl.kernel,
        mesh=_mesh,
        out_type=jax.ShapeDtypeStruct((NC, RPAD, DPAD), jnp.float32),
        scratch_types=[
            pltpu.VMEM((NCHUNK, CHUNK), jnp.int32),
            pltpu.VMEM((CHUNK, DPAD), jnp.float32),
            pltpu.VMEM_SHARED((RPAD, DPAD), jnp.float32),
        ],
        compiler_params=pltpu.CompilerParams(use_tc_tiling_on_sc=False),
    )
    def deg_kernel(dsts_hbm, zeros_hbm, ones_hbm, out_hbm, idx_d, rows, acc):
        c = lax.axis_index("c")
        s = lax.axis_index("s")
        w = s * NC + c
        pltpu.sync_copy(dsts_hbm.at[w], idx_d)
        pltpu.sync_copy(zeros_hbm, rows)
        for k in range(RPW // CHUNK):
            pltpu.sync_copy(rows, acc.at[pl.ds(s * RPW + k * CHUNK, CHUNK)])
        plsc.subcore_barrier()
        pltpu.sync_copy(ones_hbm, rows)

        def body(j, carry):
            pltpu.sync_copy(rows, acc.at[idx_d.at[j]], add=True)
            return carry

        lax.fori_loop(0, NCHUNK, body, 0)
        plsc.subcore_barrier()
        pltpu.sync_copy(acc.at[pl.ds(s * RPW, RPW)],
                        out_hbm.at[c, pl.ds(s * RPW, RPW)])

    return deg_kernel


def _make_agg_kernel(D):
    @functools.partial(
        pl.kernel,
        mesh=_mesh,
        out_type=jax.ShapeDtypeStruct((NC, RPAD, D), jnp.float32),
        scratch_types=[
            pltpu.VMEM((NCHUNK, CHUNK), jnp.int32),
            pltpu.VMEM((NCHUNK, CHUNK), jnp.int32),
            pltpu.VMEM((CHUNK, D), jnp.float32),
            pltpu.VMEM_SHARED((RPAD, D), jnp.float32),
            pltpu.SemaphoreType.DMA,
        ],
        compiler_params=(None if D % 128 == 0 else
                         pltpu.CompilerParams(use_tc_tiling_on_sc=False)),
    )
    def agg_kernel(table_hbm, srcs_hbm, dsts_hbm, zeros_hbm, out_hbm,
                   idx_s, idx_d, rows, acc, sem_g):
        c = lax.axis_index("c")
        s = lax.axis_index("s")
        w = s * NC + c
        pltpu.sync_copy(srcs_hbm.at[w], idx_s)
        pltpu.sync_copy(dsts_hbm.at[w], idx_d)
        pltpu.sync_copy(zeros_hbm, rows)
        for k in range(RPW // CHUNK):
            pltpu.sync_copy(rows, acc.at[pl.ds(s * RPW + k * CHUNK, CHUNK)])
        plsc.subcore_barrier()

        def body(j, carry):
            pltpu.async_copy(table_hbm.at[idx_s.at[j]], rows, sem_g).wait()
            pltpu.sync_copy(rows, acc.at[idx_d.at[j]], add=True)
            return carry

        lax.fori_loop(0, NCHUNK, body, 0)
        plsc.subcore_barrier()
        pltpu.sync_copy(acc.at[pl.ds(s * RPW, RPW)],
                        out_hbm.at[c, pl.ds(s * RPW, RPW)])

    return agg_kernel


_deg_kernel = _make_deg_kernel()
_agg128 = _make_agg_kernel(F1)
_agg8 = _make_agg_kernel(DPAD)

BR = 400
GRID = N // BR


def _tc1_body(deg_ref, x_ref, w_ref, h_ref, n_ref):
    deg = deg_ref[0, :, 0:1] + deg_ref[1, :, 0:1]
    norm = lax.rsqrt(jnp.maximum(deg, 1.0))
    h = jnp.dot(x_ref[...], w_ref[...], preferred_element_type=jnp.float32)
    h_ref[...] = h * norm
    n_ref[...] = norm


def _tc2_body(p_ref, n_ref, b_ref, w_ref, z_ref):
    norm = n_ref[...]
    h1 = jnp.maximum((p_ref[0] + p_ref[1]) * norm + b_ref[...], 0.0)
    z = jnp.dot(h1, w_ref[...], preferred_element_type=jnp.float32)
    z_ref[...] = z * norm


def _tc3_body(q_ref, n_ref, b_ref, o_ref):
    logits = (q_ref[0, :, 0:F2] + q_ref[1, :, 0:F2]) * n_ref[...] + b_ref[...]
    m = jnp.max(logits, axis=1, keepdims=True)
    e = jnp.exp(logits - m)
    o_ref[...] = e / jnp.sum(e, axis=1, keepdims=True)


def kernel(x, edge_index, W0, b0, W1, b1):
    ei = edge_index.astype(jnp.int32)
    npad = NW * EW - E
    pad_dst = PAD_DST + jnp.arange(npad, dtype=jnp.int32) % (RPAD - N)
    src = jnp.concatenate([ei[0], jnp.zeros((npad,), jnp.int32)])
    dst = jnp.concatenate([ei[1], pad_dst])
    srcs = src.reshape(NW, NCHUNK, CHUNK)
    dsts = dst.reshape(NW, NCHUNK, CHUNK)

    zeros8 = jnp.zeros((CHUNK, DPAD), jnp.float32)
    ones8 = jnp.ones((CHUNK, DPAD), jnp.float32)
    zeros128 = jnp.zeros((CHUNK, F1), jnp.float32)
    W1p = jnp.pad(W1, ((0, 0), (0, DPAD - F2)))

    deg_p = _deg_kernel(dsts, zeros8, ones8)

    h_scaled, norm = pl.pallas_call(
        _tc1_body,
        grid=(GRID,),
        in_specs=[
            pl.BlockSpec((NC, BR, DPAD), lambda i: (0, i, 0)),
            pl.BlockSpec((BR, F0), lambda i: (i, 0)),
            pl.BlockSpec((F0, F1), lambda i: (0, 0)),
        ],
        out_specs=[
            pl.BlockSpec((BR, F1), lambda i: (i, 0)),
            pl.BlockSpec((BR, 1), lambda i: (i, 0)),
        ],
        out_shape=[
            jax.ShapeDtypeStruct((N, F1), jnp.float32),
            jax.ShapeDtypeStruct((N, 1), jnp.float32),
        ],
    )(deg_p, x, W0)

    p = _agg128(h_scaled, srcs, dsts, zeros128)

    z_scaled = pl.pallas_call(
        _tc2_body,
        grid=(GRID,),
        in_specs=[
            pl.BlockSpec((NC, BR, F1), lambda i: (0, i, 0)),
            pl.BlockSpec((BR, 1), lambda i: (i, 0)),
            pl.BlockSpec((1, F1), lambda i: (0, 0)),
            pl.BlockSpec((F0, DPAD), lambda i: (0, 0)),
        ],
        out_specs=pl.BlockSpec((BR, DPAD), lambda i: (i, 0)),
        out_shape=jax.ShapeDtypeStruct((N, DPAD), jnp.float32),
    )(p, norm, b0.reshape(1, F1), W1p)

    q = _agg8(z_scaled, srcs, dsts, zeros8)

    out = pl.pallas_call(
        _tc3_body,
        grid=(GRID,),
        in_specs=[
            pl.BlockSpec((NC, BR, DPAD), lambda i: (0, i, 0)),
            pl.BlockSpec((BR, 1), lambda i: (i, 0)),
            pl.BlockSpec((1, F2), lambda i: (0, 0)),
        ],
        out_specs=pl.BlockSpec((BR, F2), lambda i: (i, 0)),
        out_shape=jax.ShapeDtypeStruct((N, F2), jnp.float32),
    )(q, norm, b1.reshape(1, F2))

    return out


# speedup vs baseline: 3.1282x; 3.1210x over previous
import functools

import jax
import jax.numpy as jnp
from jax import lax
from jax.experimental import pallas as pl
from jax.experimental.pallas import tpu as pltpu
from jax.experimental.pallas import tpu_sc as plsc

N = 10000
E = 320000
F0, F1, F2 = 128, 128, 2

NC = 2
NS = 16
NW = NC * NS
CHUNK = 128
NBUF = 2
GB = 16
NCHUNK = 80
NBATCH = NCHUNK // GB
EW = NCHUNK * CHUNK
RPAD = 10240
RPW = RPAD // NS
PAD_DST = N
DPAD = 8

_mesh = plsc.VectorSubcoreMesh(core_axis_name="c", subcore_axis_name="s")


def _make_deg_kernel():
    @functools.partial(
        pl.kernel,
        mesh=_mesh,
        out_type=jax.ShapeDtypeStruct((NC, RPAD, DPAD), jnp.float32),
        scratch_types=[
            pltpu.VMEM((NCHUNK, CHUNK), jnp.int32),
            pltpu.VMEM((CHUNK, DPAD), jnp.float32),
            pltpu.VMEM_SHARED((RPAD, DPAD), jnp.float32),
            pltpu.SemaphoreType.DMA,
        ],
        compiler_params=pltpu.CompilerParams(use_tc_tiling_on_sc=False),
    )
    def deg_kernel(dsts_hbm, zeros_hbm, ones_hbm, out_hbm, idx_d, rows, acc,
                   sem_s):
        c = lax.axis_index("c")
        s = lax.axis_index("s")
        w = s * NC + c
        pltpu.sync_copy(dsts_hbm.at[w], idx_d)
        pltpu.sync_copy(zeros_hbm, rows)
        for k in range(RPW // CHUNK):
            pltpu.sync_copy(rows, acc.at[pl.ds(s * RPW + k * CHUNK, CHUNK)])
        plsc.subcore_barrier()
        pltpu.sync_copy(ones_hbm, rows)

        def scatter(j):
            return pltpu.make_async_copy(rows, acc.at[idx_d.at[j]], sem_s)

        def body(bg, carry):
            for k in range(GB):
                scatter(bg * GB + k).start(add=True)
            for k in range(GB):
                scatter(bg * GB + k).wait()
            return carry

        lax.fori_loop(0, NBATCH, body, 0)
        plsc.subcore_barrier()
        pltpu.sync_copy(acc.at[pl.ds(s * RPW, RPW)],
                        out_hbm.at[c, pl.ds(s * RPW, RPW)])

    return deg_kernel


def _make_agg_kernel(D):
    nbuf = NBUF if D >= 128 else 4
    @functools.partial(
        pl.kernel,
        mesh=_mesh,
        out_type=jax.ShapeDtypeStruct((NC, RPAD, D), jnp.float32),
        scratch_types=[
            pltpu.VMEM((NCHUNK, CHUNK), jnp.int32),
            pltpu.VMEM((GB, CHUNK), jnp.int32),
            pltpu.VMEM((nbuf, CHUNK, D), jnp.float32),
            pltpu.VMEM_SHARED((RPAD, D), jnp.float32),
            pltpu.SemaphoreType.DMA,
        ],
        compiler_params=(None if D % 128 == 0 else
                         pltpu.CompilerParams(use_tc_tiling_on_sc=False)),
    )
    def agg_kernel(table_hbm, srcs_hbm, dsts_hbm, zeros_hbm, out_hbm,
                   idx_s, ib_d, rows, acc, sem_g):
        c = lax.axis_index("c")
        s = lax.axis_index("s")
        w = s * NC + c
        pltpu.sync_copy(srcs_hbm.at[w], idx_s)
        pltpu.sync_copy(zeros_hbm, rows.at[0])
        for k in range(RPW // CHUNK):
            pltpu.sync_copy(rows.at[0], acc.at[pl.ds(s * RPW + k * CHUNK, CHUNK)])
        plsc.subcore_barrier()

        def gather(j, b):
            return pltpu.make_async_copy(table_hbm.at[idx_s.at[j]],
                                         rows.at[b], sem_g)

        for b in range(nbuf):
            gather(b, b).start()

        def body(bg, carry):
            pltpu.sync_copy(dsts_hbm.at[w, pl.ds(bg * GB, GB)], ib_d)
            for k in range(GB):
                b = k % nbuf
                j = bg * GB + k
                gather(j, b).wait()
                pltpu.sync_copy(rows.at[b], acc.at[ib_d.at[k]], add=True)

                @pl.when(j + nbuf < NCHUNK)
                def _():
                    gather(j + nbuf, b).start()

            return carry

        lax.fori_loop(0, NBATCH, body, 0)
        plsc.subcore_barrier()
        pltpu.sync_copy(acc.at[pl.ds(s * RPW, RPW)],
                        out_hbm.at[c, pl.ds(s * RPW, RPW)])

    return agg_kernel


_deg_kernel = _make_deg_kernel()
_agg128 = _make_agg_kernel(F1)
_agg8 = _make_agg_kernel(DPAD)

BR = 400
GRID = N // BR


def _tc1_body(deg_ref, x_ref, w_ref, h_ref, n_ref):
    deg = deg_ref[0, :, 0:1] + deg_ref[1, :, 0:1]
    norm = lax.rsqrt(jnp.maximum(deg, 1.0))
    h = jnp.dot(x_ref[...], w_ref[...], preferred_element_type=jnp.float32)
    h_ref[...] = h * norm
    n_ref[...] = norm


def _tc2_body(p_ref, n_ref, b_ref, w_ref, z_ref):
    norm = n_ref[...]
    h1 = jnp.maximum((p_ref[0] + p_ref[1]) * norm + b_ref[...], 0.0)
    z = jnp.dot(h1, w_ref[...], preferred_element_type=jnp.float32)
    z_ref[...] = z * norm


def _tc3_body(q_ref, n_ref, b_ref, o_ref):
    logits = (q_ref[0, :, 0:F2] + q_ref[1, :, 0:F2]) * n_ref[...] + b_ref[...]
    m = jnp.max(logits, axis=1, keepdims=True)
    e = jnp.exp(logits - m)
    o_ref[...] = e / jnp.sum(e, axis=1, keepdims=True)


def kernel(x, edge_index, W0, b0, W1, b1):
    ei = edge_index.astype(jnp.int32)
    npad = NW * EW - E
    pad_iota = jnp.arange(npad, dtype=jnp.int32)
    pad_src = pad_iota % N
    pad_dst = PAD_DST + pad_iota % (RPAD - N)
    src = jnp.concatenate([ei[0], pad_src])
    dst = jnp.concatenate([ei[1], pad_dst])
    srcs = src.reshape(NW, NCHUNK, CHUNK)
    dsts = dst.reshape(NW, NCHUNK, CHUNK)

    zeros8 = jnp.zeros((CHUNK, DPAD), jnp.float32)
    ones8 = jnp.ones((CHUNK, DPAD), jnp.float32)
    zeros128 = jnp.zeros((CHUNK, F1), jnp.float32)
    W1p = jnp.pad(W1, ((0, 0), (0, DPAD - F2)))

    deg_p = _deg_kernel(dsts, zeros8, ones8)

    h_scaled, norm = pl.pallas_call(
        _tc1_body,
        grid=(GRID,),
        in_specs=[
            pl.BlockSpec((NC, BR, DPAD), lambda i: (0, i, 0)),
            pl.BlockSpec((BR, F0), lambda i: (i, 0)),
            pl.BlockSpec((F0, F1), lambda i: (0, 0)),
        ],
        out_specs=[
            pl.BlockSpec((BR, F1), lambda i: (i, 0)),
            pl.BlockSpec((BR, 1), lambda i: (i, 0)),
        ],
        out_shape=[
            jax.ShapeDtypeStruct((N, F1), jnp.float32),
            jax.ShapeDtypeStruct((N, 1), jnp.float32),
        ],
    )(deg_p, x, W0)

    p = _agg128(h_scaled, srcs, dsts, zeros128)

    z_scaled = pl.pallas_call(
        _tc2_body,
        grid=(GRID,),
        in_specs=[
            pl.BlockSpec((NC, BR, F1), lambda i: (0, i, 0)),
            pl.BlockSpec((BR, 1), lambda i: (i, 0)),
            pl.BlockSpec((1, F1), lambda i: (0, 0)),
            pl.BlockSpec((F0, DPAD), lambda i: (0, 0)),
        ],
        out_specs=pl.BlockSpec((BR, DPAD), lambda i: (i, 0)),
        out_shape=jax.ShapeDtypeStruct((N, DPAD), jnp.float32),
    )(p, norm, b0.reshape(1, F1), W1p)

    q = _agg8(z_scaled, srcs, dsts, zeros8)

    out = pl.pallas_call(
        _tc3_body,
        grid=(GRID,),
        in_specs=[
            pl.BlockSpec((NC, BR, DPAD), lambda i: (0, i, 0)),
            pl.BlockSpec((BR, 1), lambda i: (i, 0)),
            pl.BlockSpec((1, F2), lambda i: (0, 0)),
        ],
        out_specs=pl.BlockSpec((BR, F2), lambda i: (i, 0)),
        out_shape=jax.ShapeDtypeStruct((N, F2), jnp.float32),
    )(q, norm, b1.reshape(1, F2))

    return out
